# baseline (speedup 1.0000x reference)
import jax
import jax.numpy as jnp
from jax.experimental import pallas as pl
from jax.experimental.pallas import tpu as pltpu

_B, _N, _C = 64, 8732, 21
_NPAD = 8832
_ONE_BITS_P1 = 0x3F800001


def _mbl_kernel(yt_ref, loc_ref, log_ref, out_ref, hard_ref, closs_ref, stats_ref):
    b = pl.program_id(0)

    @pl.when(b == 0)
    def _init_pads():
        hard_ref[:, pl.ds(_N, _NPAD - _N)] = jnp.full(
            (_B, _NPAD - _N), -1.0, jnp.float32)
        closs_ref[:, pl.ds(_N, _NPAD - _N)] = jnp.zeros(
            (_B, _NPAD - _N), jnp.float32)

    yt = yt_ref[0]
    lt = yt[0:4].astype(jnp.float32)
    ct = yt[4:25].astype(jnp.float32)
    m = yt[25:26].astype(jnp.float32)
    x = log_ref[0].astype(jnp.float32)

    mx = jnp.max(x, axis=0, keepdims=True)
    ex = jnp.exp(x - mx)
    se = jnp.sum(ex, axis=0, keepdims=True)
    logp = (x - mx) - jnp.log(se)
    closs = -jnp.sum(jnp.maximum(ct, 1e-7) * logp, axis=0, keepdims=True)

    yp = loc_ref[0].astype(jnp.float32)
    diff = jnp.abs(lt - yp)
    l1 = jnp.where(diff < 1.0, 0.5 * diff * diff, diff - 0.5)
    lloss = jnp.sum(l1, axis=0, keepdims=True)

    np_s = jnp.sum(m)
    pc_s = jnp.sum(closs * m)
    pll_s = jnp.sum(lloss * m)
    hard = ((se - ex[0:1]) / se) * (1.0 - m)

    hard_ref[pl.ds(b, 1), pl.ds(0, _N)] = hard
    closs_ref[pl.ds(b, 1), pl.ds(0, _N)] = closs

    lane = jax.lax.broadcasted_iota(jnp.int32, (1, 128), 1)
    row = jnp.where(lane == 0, np_s,
                    jnp.where(lane == 1, pc_s,
                              jnp.where(lane == 2, pll_s, 0.0)))
    stats_ref[pl.ds(b, 1), :] = row

    @pl.when(b == _B - 1)
    def _selection():
        bits = jax.lax.bitcast_convert_type(hard_ref[...], jnp.int32)
        npv = stats_ref[:, 0:1]
        nn = jnp.minimum(3.0 * npv, float(_N) - npv)
        has = jnp.sum((nn > 0.0).astype(jnp.float32))
        kf = jnp.where(has > 0.0, jnp.sum(nn), 100.0)
        k = kf.astype(jnp.int32)

        def vbody(_, carry):
            lo, hi = carry
            mid = lo + (hi - lo) // 2
            c = jnp.sum((bits >= mid).astype(jnp.int32))
            keep = c >= k
            return (jnp.where(keep, mid, lo), jnp.where(keep, hi, mid))

        lo, _hi = jax.lax.fori_loop(
            0, 31, vbody, (jnp.int32(-1), jnp.int32(_ONE_BITS_P1)))
        t = lo
        cl = closs_ref[...]
        cnt_ge = jnp.sum((bits >= t).astype(jnp.int32))

        def _no_ties(_):
            return jnp.sum(jnp.where(bits >= t, cl, 0.0))

        def _with_ties(_):
            gt = bits > t
            tied = bits == t
            cnt_gt = jnp.sum(gt.astype(jnp.int32))
            r = k - cnt_gt
            fi = (jax.lax.broadcasted_iota(jnp.int32, (_B, _NPAD), 0) * _N
                  + jax.lax.broadcasted_iota(jnp.int32, (_B, _NPAD), 1))

            def ibody(_, carry):
                lo_i, hi_i = carry
                mid = lo_i + (hi_i - lo_i) // 2
                c = jnp.sum((tied & (fi < mid)).astype(jnp.int32))
                ge = c >= r
                return (jnp.where(ge, lo_i, mid), jnp.where(ge, mid, hi_i))

            _lo_i, m_i = jax.lax.fori_loop(
                0, 20, ibody, (jnp.int32(0), jnp.int32(_B * _N)))
            return (jnp.sum(jnp.where(gt, cl, 0.0))
                    + jnp.sum(jnp.where(tied & (fi < m_i), cl, 0.0)))

        neg = jax.lax.cond(cnt_ge == k, _no_ties, _with_ties, 0)
        denom = jnp.sum(jnp.where(npv != 0.0, npv, 1.0))
        pc_t = jnp.sum(stats_ref[:, 1:2])
        pl_t = jnp.sum(stats_ref[:, 2:3])
        out_ref[...] = ((pc_t + neg + pl_t) / denom).reshape(1, 1)


def _build_call(interpret=False):
    return pl.pallas_call(
        _mbl_kernel,
        grid=(_B,),
        in_specs=[
            pl.BlockSpec((1, 26, _N), lambda b: (b, 0, 0)),
            pl.BlockSpec((1, 4, _N), lambda b: (b, 0, 0)),
            pl.BlockSpec((1, _C, _N), lambda b: (b, 0, 0)),
        ],
        out_specs=pl.BlockSpec((1, 1), lambda b: (0, 0)),
        out_shape=jax.ShapeDtypeStruct((1, 1), jnp.float32),
        scratch_shapes=[
            pltpu.VMEM((_B, _NPAD), jnp.float32),
            pltpu.VMEM((_B, _NPAD), jnp.float32),
            pltpu.VMEM((_B, 128), jnp.float32),
        ],
        compiler_params=pltpu.CompilerParams(
            dimension_semantics=("arbitrary",),
            vmem_limit_bytes=100 * 1024 * 1024,
        ),
        interpret=interpret,
    )


@jax.jit
def kernel(y_true, y_pred_loc, y_pred_logits):
    ytt = jnp.swapaxes(y_true.astype(jnp.bfloat16), 1, 2)
    loct = jnp.swapaxes(y_pred_loc.astype(jnp.bfloat16), 1, 2)
    logt = jnp.swapaxes(y_pred_logits.astype(jnp.bfloat16), 1, 2)
    out = _build_call()(ytt, loct, logt)
    return out[0, 0]


# speedup vs baseline: 1.0405x; 1.0405x over previous
import jax
import jax.numpy as jnp
from jax.experimental import pallas as pl
from jax.experimental.pallas import tpu as pltpu

_B, _N, _C = 64, 8732, 21
_NPAD = 8832
_ONE_BITS_P1 = 0x3F800001


def _mbl_kernel(yt_ref, loc_ref, log_ref, out_ref, hard_ref, closs_ref, stats_ref):
    b = pl.program_id(0)

    @pl.when(b == 0)
    def _init_pads():
        hard_ref[:, pl.ds(_N, _NPAD - _N)] = jnp.full(
            (_B, _NPAD - _N), -1.0, jnp.float32)
        closs_ref[:, pl.ds(_N, _NPAD - _N)] = jnp.zeros(
            (_B, _NPAD - _N), jnp.float32)

    yt = yt_ref[0]
    lt = yt[0:4]
    ct = yt[4:25]
    m = yt[25:26]
    x = log_ref[0]

    mx = jnp.max(x, axis=0, keepdims=True)
    ex = jnp.exp(x - mx)
    se = jnp.sum(ex, axis=0, keepdims=True)
    logp = (x - mx) - jnp.log(se)
    closs = -jnp.sum(jnp.maximum(ct, 1e-7) * logp, axis=0, keepdims=True)

    yp = loc_ref[0]
    diff = jnp.abs(lt - yp)
    l1 = jnp.where(diff < 1.0, 0.5 * diff * diff, diff - 0.5)
    lloss = jnp.sum(l1, axis=0, keepdims=True)

    np_s = jnp.sum(m)
    pc_s = jnp.sum(closs * m)
    pll_s = jnp.sum(lloss * m)
    hard = ((se - ex[0:1]) / se) * (1.0 - m)

    hard_ref[pl.ds(b, 1), pl.ds(0, _N)] = hard
    closs_ref[pl.ds(b, 1), pl.ds(0, _N)] = closs

    lane = jax.lax.broadcasted_iota(jnp.int32, (1, 128), 1)
    row = jnp.where(lane == 0, np_s,
                    jnp.where(lane == 1, pc_s,
                              jnp.where(lane == 2, pll_s, 0.0)))
    stats_ref[pl.ds(b, 1), :] = row

    @pl.when(b == _B - 1)
    def _selection():
        bits = jax.lax.bitcast_convert_type(hard_ref[...], jnp.int32)
        npv = stats_ref[:, 0:1]
        nn = jnp.minimum(3.0 * npv, float(_N) - npv)
        has = jnp.sum((nn > 0.0).astype(jnp.float32))
        kf = jnp.where(has > 0.0, jnp.sum(nn), 100.0)
        k = kf.astype(jnp.int32)

        def vbody(_, carry):
            lo, hi = carry
            mid = lo + (hi - lo) // 2
            c = jnp.sum((bits >= mid).astype(jnp.int32))
            keep = c >= k
            return (jnp.where(keep, mid, lo), jnp.where(keep, hi, mid))

        lo, _hi = jax.lax.fori_loop(
            0, 31, vbody, (jnp.int32(-1), jnp.int32(_ONE_BITS_P1)))
        t = lo
        cl = closs_ref[...]
        cnt_ge = jnp.sum((bits >= t).astype(jnp.int32))

        def _no_ties(_):
            return jnp.sum(jnp.where(bits >= t, cl, 0.0))

        def _with_ties(_):
            gt = bits > t
            tied = bits == t
            cnt_gt = jnp.sum(gt.astype(jnp.int32))
            r = k - cnt_gt
            fi = (jax.lax.broadcasted_iota(jnp.int32, (_B, _NPAD), 0) * _N
                  + jax.lax.broadcasted_iota(jnp.int32, (_B, _NPAD), 1))

            def ibody(_, carry):
                lo_i, hi_i = carry
                mid = lo_i + (hi_i - lo_i) // 2
                c = jnp.sum((tied & (fi < mid)).astype(jnp.int32))
                ge = c >= r
                return (jnp.where(ge, lo_i, mid), jnp.where(ge, mid, hi_i))

            _lo_i, m_i = jax.lax.fori_loop(
                0, 20, ibody, (jnp.int32(0), jnp.int32(_B * _N)))
            return (jnp.sum(jnp.where(gt, cl, 0.0))
                    + jnp.sum(jnp.where(tied & (fi < m_i), cl, 0.0)))

        neg = jax.lax.cond(cnt_ge == k, _no_ties, _with_ties, 0)
        denom = jnp.sum(jnp.where(npv != 0.0, npv, 1.0))
        pc_t = jnp.sum(stats_ref[:, 1:2])
        pl_t = jnp.sum(stats_ref[:, 2:3])
        out_ref[...] = ((pc_t + neg + pl_t) / denom).reshape(1, 1)


def _build_call(interpret=False):
    return pl.pallas_call(
        _mbl_kernel,
        grid=(_B,),
        in_specs=[
            pl.BlockSpec((1, 26, _N), lambda b: (b, 0, 0)),
            pl.BlockSpec((1, 4, _N), lambda b: (b, 0, 0)),
            pl.BlockSpec((1, _C, _N), lambda b: (b, 0, 0)),
        ],
        out_specs=pl.BlockSpec((1, 1), lambda b: (0, 0)),
        out_shape=jax.ShapeDtypeStruct((1, 1), jnp.float32),
        scratch_shapes=[
            pltpu.VMEM((_B, _NPAD), jnp.float32),
            pltpu.VMEM((_B, _NPAD), jnp.float32),
            pltpu.VMEM((_B, 128), jnp.float32),
        ],
        compiler_params=pltpu.CompilerParams(
            dimension_semantics=("arbitrary",),
            vmem_limit_bytes=100 * 1024 * 1024,
        ),
        interpret=interpret,
    )


@jax.jit
def kernel(y_true, y_pred_loc, y_pred_logits):
    ytt = jnp.swapaxes(y_true, 1, 2)
    loct = jnp.swapaxes(y_pred_loc, 1, 2)
    logt = jnp.swapaxes(y_pred_logits, 1, 2)
    out = _build_call()(ytt, loct, logt)
    return out[0, 0]
